# Initial kernel scaffold; baseline (speedup 1.0000x reference)
import jax
import jax.numpy as jnp
from jax.experimental import pallas as pl


def kernel(pointcloud, params):
    raise NotImplementedError("write your pallas kernel here")


# speedup vs baseline: 10.6020x; 10.6020x over previous
import functools

import jax
import jax.numpy as jnp
from jax import lax
from jax.experimental import pallas as pl
from jax.experimental.pallas import tpu as pltpu
from jax.experimental.pallas import tpu_sc as plsc

_NC = 2
_NS = 16
_NW = _NC * _NS
_L = 16


def _fps_kernel(P, N, x_ref, y_ref, z_ref, idx_ref, cx_ref, cy_ref, cz_ref):
    R = N // 128
    Pr = P // 128
    x = x_ref[0]
    y = y_ref[0]
    z = z_ref[0]
    iota_n = (lax.broadcasted_iota(jnp.int32, (R, 128), 0) * 128
              + lax.broadcasted_iota(jnp.int32, (R, 128), 1))
    iota_p = (lax.broadcasted_iota(jnp.int32, (Pr, 128), 0) * 128
              + lax.broadcasted_iota(jnp.int32, (Pr, 128), 1))

    def body(i, state):
        dist, far, idxs, cxs, cys, czs = state
        sel_n = iota_n == far
        cx = jnp.sum(jnp.where(sel_n, x, 0.0))
        cy = jnp.sum(jnp.where(sel_n, y, 0.0))
        cz = jnp.sum(jnp.where(sel_n, z, 0.0))
        sel_p = iota_p == i
        idxs = jnp.where(sel_p, far, idxs)
        cxs = jnp.where(sel_p, cx, cxs)
        cys = jnp.where(sel_p, cy, cys)
        czs = jnp.where(sel_p, cz, czs)
        dx = x - cx
        dy = y - cy
        dz = z - cz
        d = dx * dx + dy * dy
        d = d + dz * dz
        dist = jnp.minimum(dist, d)
        m = jnp.max(dist)
        far = jnp.min(jnp.where(dist == m, iota_n, jnp.int32(N)))
        return dist, far, idxs, cxs, cys, czs

    init = (jnp.full((R, 128), 1e10, jnp.float32), jnp.int32(0),
            jnp.zeros((Pr, 128), jnp.int32), jnp.zeros((Pr, 128), jnp.float32),
            jnp.zeros((Pr, 128), jnp.float32), jnp.zeros((Pr, 128), jnp.float32))
    _, _, idxs, cxs, cys, czs = lax.fori_loop(0, P, body, init)
    idx_ref[0] = idxs
    cx_ref[0] = cxs
    cy_ref[0] = cys
    cz_ref[0] = czs


def _fps_call(X, Y, Z, P):
    B, N = X.shape
    R = N // 128
    Pr = P // 128
    xb = X.reshape(B, R, 128)
    yb = Y.reshape(B, R, 128)
    zb = Z.reshape(B, R, 128)
    in_spec = pl.BlockSpec((1, R, 128), lambda b: (b, 0, 0))
    out_spec = pl.BlockSpec((1, Pr, 128), lambda b: (b, 0, 0))
    out_shape = [jax.ShapeDtypeStruct((B, Pr, 128), jnp.int32)] + \
        [jax.ShapeDtypeStruct((B, Pr, 128), jnp.float32)] * 3
    idx, cx, cy, cz = pl.pallas_call(
        functools.partial(_fps_kernel, P, N),
        grid=(B,),
        in_specs=[in_spec] * 3,
        out_specs=[out_spec] * 4,
        out_shape=out_shape,
    )(xb, yb, zb)
    return (idx.reshape(B, P), cx.reshape(B, P), cy.reshape(B, P),
            cz.reshape(B, P))


def _table_kernel(inv_r, xyz_ref, wx_ref, f_ref, wf_ref, t_ref):
    t = jnp.dot(xyz_ref[0], wx_ref[...], preferred_element_type=jnp.float32)
    t = t * inv_r
    if f_ref is not None:
        t = t + jnp.dot(f_ref[0], wf_ref[...],
                        preferred_element_type=jnp.float32)
    t_ref[0] = t


def _table_call(xyz, wx, feats, wf, inv_r):
    B, N, _ = xyz.shape
    C1 = wx.shape[1]
    in_specs = [pl.BlockSpec((1, N, 3), lambda b: (b, 0, 0)),
                pl.BlockSpec(wx.shape, lambda b: (0, 0))]
    args = [xyz, wx]
    if feats is not None:
        C = feats.shape[2]
        in_specs += [pl.BlockSpec((1, N, C), lambda b: (b, 0, 0)),
                     pl.BlockSpec(wf.shape, lambda b: (0, 0))]
        args += [feats, wf]
        kern = functools.partial(_table_kernel, inv_r)
    else:
        kern = lambda x, w, o: _table_kernel(inv_r, x, w, None, None, o)
    return pl.pallas_call(
        kern,
        grid=(B,),
        in_specs=in_specs,
        out_specs=pl.BlockSpec((1, N, C1), lambda b: (b, 0, 0)),
        out_shape=jax.ShapeDtypeStruct((B, N, C1), jnp.float32),
    )(*args)


def _ballquery_call(xt, yt, zt, cx, cy, cz, N, S, ns, r2):
    BN = xt.shape[0]
    BS = cx.shape[0]
    cpt = BS // _NW
    nv = N // _L
    r2 = jnp.float32(r2)

    mesh = plsc.VectorSubcoreMesh(core_axis_name="c", subcore_axis_name="s",
                                  num_cores=_NC, num_subcores=_NS)

    def bf16r(v):
        u = plsc.bitcast(v, jnp.int32)
        u = u + jnp.int32(0x7FFF) + ((u >> 16) & 1)
        u = u & jnp.int32(-65536)
        return plsc.bitcast(u, jnp.float32)

    @functools.partial(
        pl.kernel, mesh=mesh,
        out_type=jax.ShapeDtypeStruct((BS * ns,), jnp.int32),
        compiler_params=pltpu.CompilerParams(needs_layout_passes=False),
        scratch_types=[
            pltpu.VMEM((N,), jnp.float32),
            pltpu.VMEM((N,), jnp.float32),
            pltpu.VMEM((N,), jnp.float32),
            pltpu.VMEM((N,), jnp.float32),
            pltpu.VMEM((cpt,), jnp.float32),
            pltpu.VMEM((cpt,), jnp.float32),
            pltpu.VMEM((cpt,), jnp.float32),
            pltpu.VMEM((cpt * ns + _L,), jnp.int32),
        ],
    )
    def bq(xt_h, yt_h, zt_h, cx_h, cy_h, cz_h, out_h,
           xv, yv, zv, ssv, cxv, cyv, czv, obuf):
        wid = lax.axis_index("s") * _NC + lax.axis_index("c")
        base_c = wid * cpt
        b = base_c // S
        goff = b * N
        pltpu.sync_copy(xt_h.at[pl.ds(goff, N)], xv)
        pltpu.sync_copy(yt_h.at[pl.ds(goff, N)], yv)
        pltpu.sync_copy(zt_h.at[pl.ds(goff, N)], zv)
        pltpu.sync_copy(cx_h.at[pl.ds(base_c, cpt)], cxv)
        pltpu.sync_copy(cy_h.at[pl.ds(base_c, cpt)], cyv)
        pltpu.sync_copy(cz_h.at[pl.ds(base_c, cpt)], czv)
        lane = lax.iota(jnp.int32, _L)
        take = lambda v, i: jnp.take_along_axis(v, i, axis=0)

        def prep(v, _):
            sl = pl.ds(v * _L, _L)
            xj = xv[sl]
            yj = yv[sl]
            zj = zv[sl]
            ssj = xj * xj + yj * yj
            ssv[sl] = ssj + zj * zj
            xv[sl] = bf16r(xj)
            yv[sl] = bf16r(yj)
            zv[sl] = bf16r(zj)
            return _

        lax.fori_loop(0, nv, prep, 0)

        def per_chunk(cc, _):
            cxc = cxv[pl.ds(cc * _L, _L)]
            cyc = cyv[pl.ds(cc * _L, _L)]
            czc = czv[pl.ds(cc * _L, _L)]

            def per_center(i16, _):
                i = cc * _L + i16
                isp = jnp.full((_L,), i16, jnp.int32)
                cxs = take(cxc, isp)
                cys = take(cyc, isp)
                czs = take(czc, isp)
                ssc = cxs * cxs + cys * cys
                ssc = ssc + czs * czs
                cxr = bf16r(cxs)
                cyr = bf16r(cys)
                czr = bf16r(czs)
                obase = i * ns

                def scan_body(v, ptr):
                    off = v * _L
                    xj = xv[pl.ds(off, _L)]
                    yj = yv[pl.ds(off, _L)]
                    zj = zv[pl.ds(off, _L)]
                    ssj = ssv[pl.ds(off, _L)]
                    dot = cxr * xj + cyr * yj
                    dot = dot + czr * zj
                    d2 = (ssc + ssj) - 2.0 * dot
                    m = jnp.logical_and(d2 <= r2, ptr < ns)
                    gidx = lane + (off + goff)
                    plsc.store_compressed(obuf.at[pl.ds(obase + ptr, _L)],
                                          gidx, mask=m)
                    ptr = ptr + jnp.sum(m.astype(jnp.int32))
                    return ptr

                count = lax.fori_loop(0, nv, scan_body, jnp.int32(0))

                fvec = obuf[pl.ds(obase, _L)]
                first = take(fvec, jnp.zeros((_L,), jnp.int32))
                first = jnp.where(count > 0, first,
                                  jnp.int32(goff + N - 1))
                for k in range(ns // _L):
                    pos = lane + (k * _L)
                    cur = obuf[pl.ds(obase + k * _L, _L)]
                    obuf[pl.ds(obase + k * _L, _L)] = jnp.where(
                        pos < count, cur, first)
                return _

            return lax.fori_loop(0, _L, per_center, _)

        lax.fori_loop(0, cpt // _L, per_chunk, 0)
        pltpu.sync_copy(obuf.at[pl.ds(0, cpt * ns)],
                        out_h.at[pl.ds(base_c * ns, cpt * ns)])

    out = bq(xt, yt, zt, cx, cy, cz)
    return out.reshape(BS, ns)


def _gather_call(table, idx):
    M, D = table.shape
    Rows = idx.shape[0]
    rpw = Rows // _NW
    k = rpw // 128
    idx2 = idx.reshape(Rows // 128, 128)

    mesh = plsc.VectorSubcoreMesh(core_axis_name="c", subcore_axis_name="s",
                                  num_cores=_NC, num_subcores=_NS)

    @functools.partial(
        pl.kernel, mesh=mesh,
        out_type=jax.ShapeDtypeStruct((Rows, D), jnp.float32),
        compiler_params=pltpu.CompilerParams(needs_layout_passes=False,
                                             use_tc_tiling_on_sc=False),
        scratch_types=[
            pltpu.VMEM((k, 128), jnp.int32),
            pltpu.VMEM((128, D), jnp.float32),
            pltpu.SemaphoreType.DMA,
        ],
    )
    def gk(t_h, i_h, o_h, idxv, buf, sem):
        wid = lax.axis_index("s") * _NC + lax.axis_index("c")
        pltpu.sync_copy(i_h.at[pl.ds(wid * k, k)], idxv)

        def body(j, _):
            pltpu.async_copy(t_h.at[idxv.at[j]], buf, sem).wait()
            pltpu.sync_copy(buf, o_h.at[pl.ds(wid * rpw + j * 128, 128)])
            return _

        lax.fori_loop(0, k, body, 0)

    return gk(table, idx2)


def _mlp_kernel(g, ns, inv_r, tg_ref, cen_ref, wx_ref, b1_ref, w2_ref, b2_ref,
                w3_ref, b3_ref, out_ref):
    C1 = wx_ref.shape[1]
    tcc = jnp.dot(cen_ref[...], wx_ref[...],
                  preferred_element_type=jnp.float32) * inv_r
    tcb = jnp.broadcast_to(tcc[:, None, :], (g, ns, C1)).reshape(g * ns, C1)
    h = jnp.maximum(tg_ref[...] - tcb + b1_ref[...], 0.0)
    h = jnp.maximum(jnp.dot(h, w2_ref[...],
                            preferred_element_type=jnp.float32)
                    + b2_ref[...], 0.0)
    h = jnp.maximum(jnp.dot(h, w3_ref[...],
                            preferred_element_type=jnp.float32)
                    + b3_ref[...], 0.0)
    C3 = h.shape[1]
    out_ref[...] = jnp.max(h.reshape(g, ns, C3), axis=1)


def _mlp_call(tg, centers, wx, b1, w2, b2, w3, b3, ns, inv_r):
    Rows, C1 = tg.shape
    BS = Rows // ns
    g = 32
    nblk = BS // g
    C3 = w3.shape[1]
    full = lambda a: pl.BlockSpec(a.shape, lambda i: (0,) * a.ndim)
    b1r = b1.reshape(1, -1)
    b2r = b2.reshape(1, -1)
    b3r = b3.reshape(1, -1)
    return pl.pallas_call(
        functools.partial(_mlp_kernel, g, ns, inv_r),
        grid=(nblk,),
        in_specs=[
            pl.BlockSpec((g * ns, C1), lambda i: (i, 0)),
            pl.BlockSpec((g, 3), lambda i: (i, 0)),
            full(wx), full(b1r), full(w2), full(b2r), full(w3), full(b3r),
        ],
        out_specs=pl.BlockSpec((g, C3), lambda i: (i, 0)),
        out_shape=jax.ShapeDtypeStruct((BS, C3), jnp.float32),
    )(tg, centers, wx, b1r, w2, b2r, w3, b3r)


def _fp_kernel(xu_ref, xk_ref, bb_ref, fu_ref, fk_ref, wa_ref, wb_ref,
               b1_ref, w2_ref, b2_ref, out_ref):
    xu = xu_ref[0]
    xk = xk_ref[0]
    n_u = xu.shape[0]
    n_k = xk.shape[0]
    contract = (((1,), (1,)), ((), ()))
    au = jnp.sum(xu * xu, axis=-1, keepdims=True)
    ak = bb_ref[0]
    xur = xu.astype(jnp.bfloat16).astype(jnp.float32)
    xkr = xk.astype(jnp.bfloat16).astype(jnp.float32)
    ab = lax.dot_general(xur, xkr, contract,
                         preferred_element_type=jnp.float32)
    d2 = jnp.maximum(au + ak - 2.0 * ab, 0.0)
    iota_k = lax.broadcasted_iota(jnp.int32, (n_u, n_k), 1)

    dm = d2
    ds_list = []
    idx_list = []
    for _ in range(3):
        mt = jnp.min(dm, axis=-1, keepdims=True)
        it = jnp.min(jnp.where(dm == mt, iota_k, jnp.int32(n_k)),
                     axis=-1, keepdims=True)
        ds_list.append(mt)
        idx_list.append(it)
        dm = jnp.where(iota_k == it, jnp.float32(3e38), dm)

    w0 = 1.0 / (ds_list[0] + 1e-8)
    w1 = 1.0 / (ds_list[1] + 1e-8)
    w2w = 1.0 / (ds_list[2] + 1e-8)
    wsum = w0 + w1
    wsum = wsum + w2w
    w0 = w0 / wsum
    w1 = w1 / wsum
    w2w = w2w / wsum
    zero = jnp.float32(0.0)
    m = jnp.where(iota_k == idx_list[0], w0, zero)
    m = m + jnp.where(iota_k == idx_list[1], w1, zero)
    m = m + jnp.where(iota_k == idx_list[2], w2w, zero)

    fkb = jnp.dot(fk_ref[0], wb_ref[...], preferred_element_type=jnp.float32)
    interp = jnp.dot(m, fkb, preferred_element_type=jnp.float32)
    h = jnp.dot(fu_ref[0], wa_ref[...], preferred_element_type=jnp.float32)
    h = jnp.maximum(h + interp + b1_ref[...], 0.0)
    h = jnp.maximum(jnp.dot(h, w2_ref[...],
                            preferred_element_type=jnp.float32)
                    + b2_ref[...], 0.0)
    out_ref[0] = h


def _fp_call(xyz_u, xyz_k, feats_u, feats_k, layers):
    (w1, b1), (w2, b2) = layers
    Cu = feats_u.shape[2]
    wa = w1[:Cu]
    wb = w1[Cu:]
    B, n_u, _ = xyz_u.shape
    n_k = xyz_k.shape[1]
    C2 = w2.shape[1]
    full = lambda a: pl.BlockSpec(a.shape, lambda i: (0,) * a.ndim)
    b1r = b1.reshape(1, -1)
    b2r = b2.reshape(1, -1)
    bb = jnp.sum(xyz_k * xyz_k, axis=-1)[:, None, :]
    blk = lambda a: pl.BlockSpec((1,) + a.shape[1:],
                                 lambda i: (i,) + (0,) * (a.ndim - 1))
    return pl.pallas_call(
        _fp_kernel,
        grid=(B,),
        in_specs=[blk(xyz_u), blk(xyz_k), blk(bb), blk(feats_u),
                  blk(feats_k), full(wa), full(wb), full(b1r), full(w2),
                  full(b2r)],
        out_specs=pl.BlockSpec((1, n_u, C2), lambda i: (i, 0, 0)),
        out_shape=jax.ShapeDtypeStruct((B, n_u, C2), jnp.float32),
    )(xyz_u, xyz_k, bb, feats_u, feats_k, wa, wb, b1r, w2, b2r)


def _sa_level(X, Y, Z, feats, layers, S, ns, radius):
    B, N = X.shape
    (w1, b1), (w2, b2), (w3, b3) = layers
    wx = w1[:3]
    wf = w1[3:] if w1.shape[0] > 3 else None
    inv_r = 1.0 / radius
    r2 = radius * radius

    fps_idx, nX, nY, nZ = _fps_call(X, Y, Z, S)

    xyz = jnp.stack([X, Y, Z], axis=-1)
    table = _table_call(xyz, wx, feats, wf, inv_r)

    idx = _ballquery_call(
        X.reshape(B * N), Y.reshape(B * N), Z.reshape(B * N),
        nX.reshape(B * S), nY.reshape(B * S), nZ.reshape(B * S),
        N, S, ns, r2)

    C1 = wx.shape[1]
    tg = _gather_call(table.reshape(B * N, C1), idx.reshape(B * S * ns))

    centers = jnp.stack([nX.reshape(B * S), nY.reshape(B * S),
                         nZ.reshape(B * S)], axis=-1)
    nf = _mlp_call(tg, centers, wx, b1, w2, b2, w3, b3, ns, inv_r)
    C3 = w3.shape[1]
    return nX, nY, nZ, nf.reshape(B, S, C3), fps_idx


def kernel(pointcloud, params):
    xyz = pointcloud[..., :3]
    B = xyz.shape[0]
    X = xyz[..., 0]
    Y = xyz[..., 1]
    Z = xyz[..., 2]

    X1, Y1, Z1, f1, i1 = _sa_level(X, Y, Z, None, params['sa1'],
                                   2048, 64, 0.2)
    X2, Y2, Z2, f2, _ = _sa_level(X1, Y1, Z1, f1, params['sa2'],
                                  1024, 32, 0.4)
    X3, Y3, Z3, f3, _ = _sa_level(X2, Y2, Z2, f2, params['sa3'],
                                  512, 16, 0.8)
    X4, Y4, Z4, f4, _ = _sa_level(X3, Y3, Z3, f3, params['sa4'],
                                  256, 16, 1.2)

    xyz2 = jnp.stack([X2, Y2, Z2], axis=-1)
    xyz3 = jnp.stack([X3, Y3, Z3], axis=-1)
    xyz4 = jnp.stack([X4, Y4, Z4], axis=-1)

    g = _fp_call(xyz3, xyz4, f3, f4, params['fp1'])
    g = _fp_call(xyz2, xyz3, f2, g, params['fp2'])

    fp2_inds = i1[:, :1024]
    return g, xyz2, fp2_inds
